# baseline (speedup 1.0000x reference)
import functools
import math

import jax
import jax.numpy as jnp
import numpy as np
from jax import lax
from jax.experimental import pallas as pl
from jax.experimental.pallas import tpu as pltpu
from jax.experimental.pallas import tpu_sc as plsc

VOCAB = 100000
HIDDEN = 1024
MAX_SEQ = 2048
BATCH = 4
SEQ = 2048

SCALE = math.sqrt(HIDDEN)

NC = 2
NS = 16
L = 16
NW = NC * NS
S_PER_W = SEQ // NW
CH = 32
N_CHUNK = S_PER_W // CH


def _sinusoidal_pe_np(max_seq_len, d):
    pos = np.arange(max_seq_len, dtype=np.float32)[:, None]
    i = np.arange(0, d, 2, dtype=np.float32)
    div = np.exp(-math.log(10000.0) * i / d)
    pe = np.zeros((max_seq_len, d), dtype=np.float32)
    pe[:, 0::2] = np.sin(pos * div)
    pe[:, 1::2] = np.cos(pos * div)
    return pe


_PE = _sinusoidal_pe_np(MAX_SEQ, HIDDEN)


def _body(ids_hbm, table_hbm, pe_hbm, out_hbm, idx_v, pe_v, rows_v, sem):
    wid = lax.axis_index("s") * NC + lax.axis_index("c")
    s0 = wid * S_PER_W
    for j in range(N_CHUNK):
        sj = s0 + j * CH
        pltpu.sync_copy(pe_hbm.at[pl.ds(sj, CH)], pe_v)
        for b in range(BATCH):
            row0 = b * SEQ + sj
            pltpu.sync_copy(ids_hbm.at[pl.ds(row0, CH)], idx_v)
            pltpu.async_copy(table_hbm.at[idx_v], rows_v, sem).wait()

            def comp_row(r, _):
                def comp_col(c, _):
                    sl = pl.ds(c * L, L)
                    rows_v[r, sl] = rows_v[r, sl] * SCALE + pe_v[r, sl]
                    return 0

                lax.fori_loop(0, HIDDEN // L, comp_col, 0)
                return 0

            lax.fori_loop(0, CH, comp_row, 0)
            pltpu.sync_copy(rows_v, out_hbm.at[pl.ds(row0, CH)])


@jax.jit
def _embed_sc(ids_flat, table, pe):
    mesh = plsc.VectorSubcoreMesh(core_axis_name="c", subcore_axis_name="s")
    k = functools.partial(
        pl.kernel,
        mesh=mesh,
        out_type=jax.ShapeDtypeStruct((BATCH * SEQ, HIDDEN), jnp.float32),
        scratch_types=[
            pltpu.VMEM((CH,), jnp.int32),
            pltpu.VMEM((CH, HIDDEN), jnp.float32),
            pltpu.VMEM((CH, HIDDEN), jnp.float32),
            pltpu.SemaphoreType.DMA,
        ],
    )(_body)
    return k(ids_flat, table, pe)


def kernel(input_ids, token_table):
    ids_flat = input_ids.reshape(-1).astype(jnp.int32)
    pe = jnp.asarray(_PE)
    out = _embed_sc(ids_flat, token_table, pe)
    return out.reshape(BATCH, SEQ, HIDDEN)
